# Initial kernel scaffold; baseline (speedup 1.0000x reference)
import jax
import jax.numpy as jnp
from jax.experimental import pallas as pl


def kernel(features, anchors, W_conv, b_conv, W_obj, b_obj, W_delta, b_delta):
    raise NotImplementedError("write your pallas kernel here")


# speedup vs baseline: 18.4038x; 18.4038x over previous
import math
import functools

import jax
import jax.numpy as jnp
from jax.experimental import pallas as pl
from jax.experimental.pallas import tpu as pltpu

B, C, H, W = 2, 256, 64, 64
A = 3
N = H * W * A
NS = 16384
PRE = 2000
PRE_PAD = 2048
POST = 1000
POST_PAD = 1024
NMS_THRESH = 0.7
IMG = 1024.0
SCALE_CLAMP = math.log(1000.0 / 16)
NEG = -1e30


def _xor_rows(a, j):
    g = a.reshape(128 // (2 * j), 2, j, 128)
    g = jnp.concatenate([g[:, 1:2], g[:, 0:1]], axis=1)
    return g.reshape(128, 128)


def _cexchange(arrs, j, want_first):
    partners = [_xor_rows(a, j) for a in arrs]
    k_a, i_a = arrs[0], arrs[1]
    k_p, i_p = partners[0], partners[1]
    p_before_a = (k_p > k_a) | ((k_p == k_a) & (i_p < i_a))
    take_p = p_before_a == want_first
    return [jnp.where(take_p, p, a) for a, p in zip(arrs, partners)]


def _bitonic_sort_desc(arrs):
    row_i = jax.lax.broadcasted_iota(jnp.int32, (128, 128), 0)
    col_i = jax.lax.broadcasted_iota(jnp.int32, (128, 128), 1)
    id_n = row_i * 128 + col_i
    id_t = col_i * 128 + row_i

    def stage(arrs, k, j, ident):
        dir_desc = (ident & k) == 0
        is_lo = (ident & j) == 0
        want_first = is_lo == dir_desc
        jr = j // 128 if j >= 128 else j
        return _cexchange(arrs, jr, want_first)

    transposed = False

    def to_norm(arrs):
        return [a.T for a in arrs]

    arrs = to_norm(arrs)
    transposed = True
    k = 2
    while k <= 128:
        j = k // 2
        while j >= 1:
            arrs = stage(arrs, k, j, id_t)
            j //= 2
        k *= 2
    while k <= NS:
        arrs = to_norm(arrs)
        j = k // 2
        while j >= 128:
            arrs = stage(arrs, k, j, id_n)
            j //= 2
        arrs = to_norm(arrs)
        while j >= 1:
            arrs = stage(arrs, k, j, id_t)
            j //= 2
        k *= 2
    return to_norm(arrs)


def _lane_prefix_incl(x16):
    lane_r = jax.lax.broadcasted_iota(jnp.int32, (128, 128), 0)
    lane_c = jax.lax.broadcasted_iota(jnp.int32, (128, 128), 1)
    tri = (lane_r <= lane_c).astype(jnp.float32)
    pref = jax.lax.dot_general(x16, tri, (((1,), (0,)), ((), ())),
                               preferred_element_type=jnp.float32)
    row_tot = pref[:, 127:128]
    r16 = jax.lax.broadcasted_iota(jnp.int32, (16, 16), 0)
    c16 = jax.lax.broadcasted_iota(jnp.int32, (16, 16), 1)
    t16 = (c16 < r16).astype(jnp.float32)
    offs = jax.lax.dot_general(t16, row_tot, (((1,), (0,)), ((), ())),
                               preferred_element_type=jnp.float32)
    return pref + offs


def _to_row(a16):
    return jnp.concatenate([a16[r:r + 1, :] for r in range(16)], axis=1)


def _to_col(a16):
    return jnp.concatenate(
        [jnp.transpose(a16[r:r + 1, :]) for r in range(16)], axis=0)


def _p1_body(lg_ref, dx_ref, dy_ref, dw_ref, dh_ref,
             a0_ref, a1_ref, a2_ref, a3_ref,
             ox1_ref, oy1_ref, ox2_ref, oy2_ref, osc_ref,
             m_ref, alive_ref):
    f32 = jnp.float32
    lg = lg_ref[0, 0, :].reshape(128, 128)
    dx = dx_ref[0, 0, :].reshape(128, 128)
    dy = dy_ref[0, 0, :].reshape(128, 128)
    dw = dw_ref[0, 0, :].reshape(128, 128)
    dh = dh_ref[0, 0, :].reshape(128, 128)
    a0 = a0_ref[0, 0, :].reshape(128, 128)
    a1 = a1_ref[0, 0, :].reshape(128, 128)
    a2 = a2_ref[0, 0, :].reshape(128, 128)
    a3 = a3_ref[0, 0, :].reshape(128, 128)
    aw = a2 - a0
    ah = a3 - a1
    ax = a0 + 0.5 * aw
    ay = a1 + 0.5 * ah
    dwc = jnp.minimum(dw, SCALE_CLAMP)
    dhc = jnp.minimum(dh, SCALE_CLAMP)
    px = dx * aw + ax
    py = dy * ah + ay
    pw = jnp.exp(dwc) * aw
    ph = jnp.exp(dhc) * ah
    x1 = jnp.clip(px - 0.5 * pw, 0.0, IMG)
    y1 = jnp.clip(py - 0.5 * ph, 0.0, IMG)
    x2 = jnp.clip(px + 0.5 * pw, 0.0, IMG)
    y2 = jnp.clip(py + 0.5 * ph, 0.0, IMG)
    idx = (jax.lax.broadcasted_iota(jnp.int32, (128, 128), 0) * 128 +
           jax.lax.broadcasted_iota(jnp.int32, (128, 128), 1))

    sv, _, sx1, sy1, sx2, sy2 = _bitonic_sort_desc([lg, idx, x1, y1, x2, y2])

    tv = sv[:16, :]
    tx1 = sx1[:16, :]
    ty1 = sy1[:16, :]
    tx2 = sx2[:16, :]
    ty2 = sy2[:16, :]

    area16 = (tx2 - tx1) * (ty2 - ty1)

    jx1, jy1, jx2, jy2, jar = map(_to_row, (tx1, ty1, tx2, ty2, area16))

    jpos = jax.lax.broadcasted_iota(jnp.int32, (128, PRE_PAD), 1)
    for t0 in range(0, PRE_PAD, 128):
        r = t0 // 128
        rx1 = jnp.transpose(tx1[r:r + 1, :])
        ry1 = jnp.transpose(ty1[r:r + 1, :])
        rx2 = jnp.transpose(tx2[r:r + 1, :])
        ry2 = jnp.transpose(ty2[r:r + 1, :])
        rar = jnp.transpose(area16[r:r + 1, :])
        ltx = jnp.maximum(rx1, jx1)
        lty = jnp.maximum(ry1, jy1)
        rbx = jnp.minimum(rx2, jx2)
        rby = jnp.minimum(ry2, jy2)
        wx = jnp.clip(rbx - ltx, 0.0, None)
        wy = jnp.clip(rby - lty, 0.0, None)
        inter = wx * wy
        iou = inter / (rar + jar - inter + 1e-9)
        ipos = t0 + jax.lax.broadcasted_iota(jnp.int32, (128, PRE_PAD), 0)
        m = (iou > NMS_THRESH) & (jpos > ipos) & (ipos < PRE) & (jpos < PRE)
        m_ref[t0:t0 + 128, :] = m.astype(f32)

    lane = jax.lax.broadcasted_iota(jnp.int32, (1, PRE_PAD), 1)
    alive_ref[:, :] = (lane < PRE).astype(f32)

    def nms_step(i, _):
        av = alive_ref[:, :]
        ai = jnp.sum(jnp.where(lane == i, av, 0.0))
        row = m_ref[pl.ds(i, 1), :]
        alive_ref[:, :] = av * (1.0 - ai * row)
        return 0

    jax.lax.fori_loop(0, PRE, nms_step, 0)

    keep_row = alive_ref[:, :]
    keep16 = jnp.concatenate(
        [keep_row[:, c0:c0 + 128] for c0 in range(0, PRE_PAD, 128)], axis=0)
    pos16 = (jax.lax.broadcasted_iota(jnp.int32, (16, 128), 0) * 128 +
             jax.lax.broadcasted_iota(jnp.int32, (16, 128), 1))
    in2000 = pos16 < PRE
    notk16 = (1.0 - keep16) * in2000.astype(f32)

    ck = _lane_prefix_incl(keep16)
    cn = _lane_prefix_incl(notk16)
    nk = jnp.sum(keep16)
    keep_b = keep16 > 0.5
    rank = jnp.where(keep_b, ck - 1.0, nk + cn - 1.0)
    rank = jnp.where(in2000, rank, 3000.0)
    kept_val = jnp.where(keep_b, tv, -1e9)

    rank_row = _to_row(rank)
    kv_row = _to_row(kept_val)
    px1, py1_, px2, py2_ = jx1, jy1, jx2, jy2
    for s0 in range(0, POST_PAD, 128):
        s_col = s0 + jax.lax.broadcasted_iota(jnp.int32, (128, 1), 0)
        oh = (rank_row == s_col.astype(f32)).astype(f32)
        o1 = jnp.sum(oh * px1, axis=1, keepdims=True)
        o2 = jnp.sum(oh * py1_, axis=1, keepdims=True)
        o3 = jnp.sum(oh * px2, axis=1, keepdims=True)
        o4 = jnp.sum(oh * py2_, axis=1, keepdims=True)
        o5 = jnp.sum(oh * kv_row, axis=1, keepdims=True)
        ox1_ref[0, 0, pl.ds(s0, 128)] = jnp.transpose(o1)[0, :]
        oy1_ref[0, 0, pl.ds(s0, 128)] = jnp.transpose(o2)[0, :]
        ox2_ref[0, 0, pl.ds(s0, 128)] = jnp.transpose(o3)[0, :]
        oy2_ref[0, 0, pl.ds(s0, 128)] = jnp.transpose(o4)[0, :]
        osc_ref[0, 0, pl.ds(s0, 128)] = jnp.transpose(o5)[0, :]


def _proposals(logits, dxs, dys, dws, dhs, a0, a1, a2, a3):
    grid = (B,)
    row_spec = pl.BlockSpec((1, 1, NS), lambda b: (b, 0, 0))
    anc_spec = pl.BlockSpec((1, 1, NS), lambda b: (0, 0, 0))
    out_spec = pl.BlockSpec((1, 1, POST_PAD), lambda b: (b, 0, 0))
    out_sh = jax.ShapeDtypeStruct((B, 1, POST_PAD), jnp.float32)
    return pl.pallas_call(
        _p1_body,
        grid=grid,
        in_specs=[row_spec] * 5 + [anc_spec] * 4,
        out_specs=[out_spec] * 5,
        out_shape=[out_sh] * 5,
        scratch_shapes=[
            pltpu.VMEM((PRE_PAD, PRE_PAD), jnp.float32),
            pltpu.VMEM((1, PRE_PAD), jnp.float32),
        ],
    )(logits.reshape(B, 1, NS), dxs.reshape(B, 1, NS), dys.reshape(B, 1, NS),
      dws.reshape(B, 1, NS), dhs.reshape(B, 1, NS),
      a0.reshape(1, 1, NS), a1.reshape(1, 1, NS), a2.reshape(1, 1, NS),
      a3.reshape(1, 1, NS))


def _conv(x, w, b):
    y = jax.lax.conv_general_dilated(x, w, (1, 1), 'SAME',
                                     dimension_numbers=('NCHW', 'OIHW', 'NCHW'))
    return y + b[None, :, None, None]


def kernel(features, anchors, W_conv, b_conv, W_obj, b_obj, W_delta, b_delta):
    t = jax.nn.relu(_conv(features, W_conv, b_conv))
    logits = _conv(t, W_obj, b_obj)
    deltas = _conv(t, W_delta, b_delta)
    logits = jnp.transpose(logits, (0, 2, 3, 1)).reshape(B, -1)
    deltas = deltas.reshape(B, A, 4, H, W)
    deltas = jnp.transpose(deltas, (0, 3, 4, 1, 2)).reshape(B, -1, 4)

    pad = NS - N
    lg = jnp.pad(logits, ((0, 0), (0, pad)), constant_values=NEG)
    dxs = jnp.pad(deltas[:, :, 0], ((0, 0), (0, pad)))
    dys = jnp.pad(deltas[:, :, 1], ((0, 0), (0, pad)))
    dws = jnp.pad(deltas[:, :, 2], ((0, 0), (0, pad)))
    dhs = jnp.pad(deltas[:, :, 3], ((0, 0), (0, pad)))
    anc = jnp.pad(anchors, ((0, pad), (0, 0)))
    a0 = anc[:, 0].reshape(1, NS)
    a1 = anc[:, 1].reshape(1, NS)
    a2 = anc[:, 2].reshape(1, NS)
    a3 = anc[:, 3].reshape(1, NS)

    ox1, oy1, ox2, oy2, osc = _proposals(lg, dxs, dys, dws, dhs, a0, a1, a2, a3)
    out = jnp.stack([ox1[:, 0, :POST], oy1[:, 0, :POST], ox2[:, 0, :POST],
                     oy2[:, 0, :POST], osc[:, 0, :POST]], axis=-1)
    return out
